# baseline (speedup 1.0000x reference)
import jax
import jax.numpy as jnp
from jax.experimental import pallas as pl
from jax.experimental.pallas import tpu as pltpu

EMB = 1000
CB = 256


def _onehot_t_block(labels_ref, out_ref):
    labs = labels_ref[:].astype(jnp.int32)
    rows = jax.lax.broadcasted_iota(jnp.int32, (EMB, CB), 0)
    out_ref[:, :] = (rows == labs[None, :]).astype(jnp.float32)


def kernel(labels):
    batch = labels.shape[0]
    grid = batch // CB
    out_t = pl.pallas_call(
        _onehot_t_block,
        grid=(grid,),
        in_specs=[pl.BlockSpec((CB,), lambda i: (i,))],
        out_specs=pl.BlockSpec((EMB, CB), lambda i: (0, i)),
        out_shape=jax.ShapeDtypeStruct((EMB, batch), jnp.float32),
        compiler_params=pltpu.CompilerParams(dimension_semantics=("parallel",)),
    )(labels)
    return out_t.T


# speedup vs baseline: 1.9762x; 1.9762x over previous
import jax
import jax.numpy as jnp
from jax.experimental import pallas as pl
from jax.experimental.pallas import tpu as pltpu

EMB = 1000
CB = 1024


def _onehot_t_block(labels_ref, out_ref):
    labs = labels_ref[:].astype(jnp.int32)
    rows = jax.lax.broadcasted_iota(jnp.int32, (EMB, CB), 0)
    out_ref[:, :] = (rows == labs[None, :]).astype(jnp.float32)


def kernel(labels):
    batch = labels.shape[0]
    grid = batch // CB
    out_t = pl.pallas_call(
        _onehot_t_block,
        grid=(grid,),
        in_specs=[pl.BlockSpec((CB,), lambda i: (i,))],
        out_specs=pl.BlockSpec((EMB, CB), lambda i: (0, i)),
        out_shape=jax.ShapeDtypeStruct((EMB, batch), jnp.float32),
        compiler_params=pltpu.CompilerParams(dimension_semantics=("parallel",)),
    )(labels)
    return out_t.T
